# Initial kernel scaffold; baseline (speedup 1.0000x reference)
import jax
import jax.numpy as jnp
from jax.experimental import pallas as pl


def kernel(va, ea, edge_index_a, node_graph_a, vs, es, edge_index_s, node_graph_s, params):
    raise NotImplementedError("write your pallas kernel here")


# speedup vs baseline: 1.2194x; 1.2194x over previous
import functools

import jax
import jax.numpy as jnp
from jax import lax
from jax.experimental import pallas as pl
from jax.experimental.pallas import tpu as pltpu
from jax.experimental.pallas import tpu_sc as plsc

N_NODES = 10000
N_GRAPHS = 256
E = 160000
E_PAD = 163840
EB = 2000
NB = E // EB
NROWS = 10016
NW = 32
CH = 128
PER_W = E_PAD // NW
N_CH = PER_W // CH
RPT = NROWS // 16
EPS = 1e-5


def _dotq(a, b):
    return lax.dot_general(a.astype(jnp.bfloat16), b.astype(jnp.bfloat16),
                           (((1,), (0,)), ((), ())),
                           preferred_element_type=jnp.float32)


def _bn_t(x, g, b):
    mu = jnp.mean(x, axis=0, keepdims=True)
    var = jnp.mean((x - mu) ** 2, axis=0, keepdims=True)
    return g * (x - mu) / jnp.sqrt(var + EPS) + b


def _pad2(w, r, c):
    return jnp.pad(w, ((0, r - w.shape[0]), (0, c - w.shape[1])))


def _pad1(v, n):
    return jnp.pad(v, (0, n - v.shape[0]))


@functools.partial(jax.jit, static_argnames=("dp",))
def _sc_gather(table, idx, dp):
    mesh = plsc.VectorSubcoreMesh(core_axis_name="c", subcore_axis_name="s")

    @functools.partial(
        pl.kernel, mesh=mesh,
        out_type=jax.ShapeDtypeStruct((E_PAD, dp), jnp.float32),
        compiler_params=pltpu.CompilerParams(use_tc_tiling_on_sc=False),
        scratch_types=[
            pltpu.VMEM((CH,), jnp.int32),
            pltpu.VMEM((CH, dp), jnp.float32),
            pltpu.SemaphoreType.DMA,
        ],
    )
    def gather_k(table_hbm, idx_hbm, out_hbm, idx_v, rows_v, sem):
        wid = lax.axis_index("s") * 2 + lax.axis_index("c")
        base = wid * PER_W

        def body(j, carry):
            off = base + j * CH
            pltpu.sync_copy(idx_hbm.at[pl.ds(off, CH)], idx_v)
            pltpu.async_copy(table_hbm.at[idx_v], rows_v, sem).wait()
            pltpu.sync_copy(rows_v, out_hbm.at[pl.ds(off, CH)])
            return carry

        lax.fori_loop(0, N_CH, body, 0)

    return gather_k(table, idx)


@functools.partial(jax.jit, static_argnames=("dp",))
def _sc_scatter_add(upd, dst, zeros, dp):
    mesh = plsc.VectorSubcoreMesh(core_axis_name="c", subcore_axis_name="s")

    @functools.partial(
        pl.kernel, mesh=mesh,
        out_type=jax.ShapeDtypeStruct((2, NROWS, dp), jnp.float32),
        compiler_params=pltpu.CompilerParams(use_tc_tiling_on_sc=False),
        scratch_types=[
            pltpu.VMEM((CH,), jnp.int32),
            pltpu.VMEM((CH, dp), jnp.float32),
            pltpu.VMEM_SHARED((NROWS, dp), jnp.float32),
            pltpu.SemaphoreType.DMA,
        ],
    )
    def scatter_k(upd_hbm, dst_hbm, zeros_hbm, out_hbm, idx_v, rows_v, acc_sh, sem):
        c = lax.axis_index("c")
        s = lax.axis_index("s")
        r0 = s * RPT
        pltpu.sync_copy(zeros_hbm.at[pl.ds(r0, RPT)], acc_sh.at[pl.ds(r0, RPT)])
        plsc.subcore_barrier()
        wid = s * 2 + c
        base = wid * PER_W

        def body(j, carry):
            off = base + j * CH
            pltpu.sync_copy(dst_hbm.at[pl.ds(off, CH)], idx_v)
            pltpu.sync_copy(upd_hbm.at[pl.ds(off, CH)], rows_v)
            pltpu.sync_copy(rows_v, acc_sh.at[idx_v], add=True)
            return carry

        lax.fori_loop(0, N_CH, body, 0)
        plsc.subcore_barrier()
        pltpu.sync_copy(acc_sh.at[pl.ds(r0, RPT)], out_hbm.at[c, pl.ds(r0, RPT)])

    return scatter_k(upd, dst, zeros)


def _embed_body(x_ref, w_ref, c_ref, out_ref):
    y = _dotq(x_ref[...], w_ref[...])
    out_ref[...] = jax.nn.silu(_bn_t(y, c_ref[0:1, :], c_ref[1:2, :]))


def _embed(x, w, consts, dp):
    return pl.pallas_call(
        _embed_body,
        out_shape=jax.ShapeDtypeStruct((N_NODES, dp), jnp.float32),
    )(x, w, consts)


def _ea_stats_body(ea_ref, out_ref, acc_ref):
    i = pl.program_id(0)

    @pl.when(i == 0)
    def _():
        acc_ref[...] = jnp.zeros_like(acc_ref)

    ea = ea_ref[...]
    s2 = lax.dot_general(ea, ea, (((0,), (0,)), ((), ())),
                         preferred_element_type=jnp.float32, precision=lax.Precision.HIGHEST)
    acc_ref[0:8, :] += s2
    acc_ref[8:9, :] += jnp.sum(ea, axis=0, keepdims=True)

    @pl.when(i == NB - 1)
    def _():
        out_ref[...] = acc_ref[...]


def _ea_stats(ea8):
    return pl.pallas_call(
        _ea_stats_body,
        grid=(NB,),
        in_specs=[pl.BlockSpec((EB, 8), lambda i: (i, 0))],
        out_specs=pl.BlockSpec((16, 8), lambda i: (0, 0)),
        out_shape=jax.ShapeDtypeStruct((16, 8), jnp.float32),
        scratch_shapes=[pltpu.VMEM((16, 8), jnp.float32)],
    )(ea8)


def _stage1_body(ea_ref, nsrc_ref, ndst_ref, w1m_ref, w2m_ref, w3m_ref,
                 w1s_ref, w2s_ref, w3s_ref, we_ref, c_ref,
                 ym_ref, ys_ref, mom_ref, acc_ref):
    i = pl.program_id(0)

    @pl.when(i == 0)
    def _():
        acc_ref[...] = jnp.zeros_like(acc_ref)

    z = _dotq(ea_ref[...], we_ref[...])
    hm = jax.nn.silu(z * c_ref[0:1, :] + c_ref[1:2, :])
    nsrc = nsrc_ref[...]
    ndst = ndst_ref[...]
    ym = (_dotq(nsrc, w1m_ref[...])
          + _dotq(ndst, w2m_ref[...])
          + _dotq(hm, w3m_ref[...]))
    ys = (_dotq(nsrc, w1s_ref[...])
          + _dotq(ndst, w2s_ref[...])
          + _dotq(hm, w3s_ref[...]))
    ym_ref[...] = ym
    ys_ref[...] = ys
    acc_ref[0:1, :] += jnp.sum(ym, axis=0, keepdims=True)
    acc_ref[1:2, :] += jnp.sum(ym * ym, axis=0, keepdims=True)
    acc_ref[2:3, :] += jnp.sum(ys, axis=0, keepdims=True)
    acc_ref[3:4, :] += jnp.sum(ys * ys, axis=0, keepdims=True)

    @pl.when(i == NB - 1)
    def _():
        mom_ref[...] = acc_ref[...]


def _stage1(ea8, nsrc, ndst, ws, we, consts, dp):
    full = lambda r, c: pl.BlockSpec((r, c), lambda i: (0, 0))
    blk = lambda: pl.BlockSpec((EB, dp), lambda i: (i, 0))
    return pl.pallas_call(
        _stage1_body,
        grid=(NB,),
        in_specs=[pl.BlockSpec((EB, 8), lambda i: (i, 0)), blk(), blk(),
                  full(dp, dp), full(dp, dp), full(dp, dp),
                  full(dp, dp), full(dp, dp), full(dp, dp),
                  full(8, dp), full(8, dp)],
        out_specs=[blk(), blk(), full(8, dp)],
        out_shape=[jax.ShapeDtypeStruct((E_PAD, dp), jnp.float32),
                   jax.ShapeDtypeStruct((E_PAD, dp), jnp.float32),
                   jax.ShapeDtypeStruct((8, dp), jnp.float32)],
        scratch_shapes=[pltpu.VMEM((8, dp), jnp.float32)],
    )(ea8, nsrc, ndst, *ws, we, consts)


def _stage2_body(ym_ref, ys_ref, mom_ref, c_ref, upd_ref):
    mom = mom_ref[...]
    inv_e = 1.0 / E
    mean_m = mom[0:1, :] * inv_e
    var_m = mom[1:2, :] * inv_e - mean_m * mean_m
    mean_s = mom[2:3, :] * inv_e
    var_s = mom[3:4, :] * inv_e - mean_s * mean_s
    am = (ym_ref[...] - mean_m) * (c_ref[0:1, :] * lax.rsqrt(var_m + EPS)) + c_ref[1:2, :]
    as_ = (ys_ref[...] - mean_s) * (c_ref[2:3, :] * lax.rsqrt(var_s + EPS)) + c_ref[3:4, :]
    upd_ref[...] = jax.nn.silu(am) * jax.nn.softplus(as_)


def _stage2(ym, ys, mom, consts, dp):
    full = lambda r, c: pl.BlockSpec((r, c), lambda i: (0, 0))
    blk = lambda: pl.BlockSpec((EB, dp), lambda i: (i, 0))
    return pl.pallas_call(
        _stage2_body,
        grid=(NB,),
        in_specs=[blk(), blk(), full(8, dp), full(8, dp)],
        out_specs=blk(),
        out_shape=jax.ShapeDtypeStruct((E_PAD, dp), jnp.float32),
    )(ym, ys, mom, consts)


def _node_update_body(acc_ref, node_ref, c_ref, out_ref):
    acc = acc_ref[...]
    agg = acc[0, 0:N_NODES, :] + acc[1, 0:N_NODES, :]
    node = node_ref[...]
    out_ref[...] = jax.nn.softplus(
        _bn_t(agg, c_ref[0:1, :], c_ref[1:2, :]) + node)


def _node_update(acc2, node, consts, dp):
    return pl.pallas_call(
        _node_update_body,
        out_shape=jax.ShapeDtypeStruct((N_NODES, dp), jnp.float32),
    )(acc2, node, consts)


def _pool_body(v_ref, gid_ref, out_ref, ssum_ref, cnt_ref):
    i = pl.program_id(0)

    @pl.when(i == 0)
    def _():
        ssum_ref[...] = jnp.zeros_like(ssum_ref)
        cnt_ref[...] = jnp.zeros_like(cnt_ref)

    oh = (gid_ref[...] == lax.broadcasted_iota(jnp.int32, (EB, N_GRAPHS), 1)
          ).astype(jnp.float32)
    ssum_ref[...] += lax.dot_general(oh, v_ref[...], (((0,), (0,)), ((), ())),
                                     preferred_element_type=jnp.float32, precision=lax.Precision.HIGHEST)
    cnt_ref[...] += lax.dot_general(oh, jnp.ones((EB, 8), jnp.float32),
                                    (((0,), (0,)), ((), ())),
                                    preferred_element_type=jnp.float32, precision=lax.Precision.HIGHEST)

    @pl.when(i == (N_NODES // EB) - 1)
    def _():
        d = out_ref.shape[1]
        out_ref[...] = ssum_ref[:, 0:d] / jnp.maximum(cnt_ref[:, 0:1], 1.0)


def _pool(v, gid2d, d, dp):
    return pl.pallas_call(
        _pool_body,
        grid=(N_NODES // EB,),
        in_specs=[pl.BlockSpec((EB, dp), lambda i: (i, 0)),
                  pl.BlockSpec((EB, 1), lambda i: (i, 0))],
        out_specs=pl.BlockSpec((N_GRAPHS, d), lambda i: (0, 0)),
        out_shape=jax.ShapeDtypeStruct((N_GRAPHS, d), jnp.float32),
        scratch_shapes=[pltpu.VMEM((N_GRAPHS, dp), jnp.float32),
                        pltpu.VMEM((N_GRAPHS, 8), jnp.float32)],
    )(v, gid2d)


def _head_body(vt_ref, watt_ref, catt_ref, w0_ref, c0_ref, w1_ref, c1_ref,
               w2_ref, c2_ref, wad_ref, cad_ref, wsi_ref, csi_ref,
               wt_ref, ct_ref, out_ref):
    vt = vt_ref[...]
    h = _bn_t(_dotq(vt, watt_ref[...]),
              catt_ref[0:1, :], catt_ref[1:2, :])
    eh = jnp.where(h > 0, h, jnp.exp(jnp.minimum(h, 0.0)) - 1.0)
    vt2 = _bn_t(eh * vt, catt_ref[2:3, :], catt_ref[3:4, :])
    x = vt2
    for w_ref, c_ref in ((w0_ref, c0_ref), (w1_ref, c1_ref), (w2_ref, c2_ref)):
        x = jax.nn.silu(_bn_t(
            _dotq(x, w_ref[...]),
            c_ref[0:1, :], c_ref[1:2, :]))
    ca = jax.nn.softmax(_bn_t(
        _dotq(x, wad_ref[...]),
        cad_ref[0:1, :], cad_ref[1:2, :]), axis=1)
    cs = jax.nn.softmax(_bn_t(
        _dotq(x, wsi_ref[...]),
        csi_ref[0:1, :], csi_ref[1:2, :]), axis=1)
    t = _dotq(x, wt_ref[...]) + ct_ref[0:1, 0:1]
    out_ref[...] = jnp.concatenate([ca, cs, t], axis=1)


def _head(vt, params):
    p = params
    catt = jnp.stack([p["atten"]["g"], p["atten"]["bt"],
                      p["atten_bn_g"], p["atten_bn_b"],
                      jnp.zeros_like(p["atten"]["g"]), jnp.zeros_like(p["atten"]["g"]),
                      jnp.zeros_like(p["atten"]["g"]), jnp.zeros_like(p["atten"]["g"])])
    args = [vt, p["atten"]["W"], catt]
    for fp in p["fcs"]:
        cw = jnp.stack([fp["g"], fp["bt"]] + [jnp.zeros_like(fp["g"])] * 6)
        args += [fp["W"], cw]
    for key in ("pred_adsb", "pred_site"):
        fp = p[key]
        cw = jnp.stack([fp["g"], fp["bt"]] + [jnp.zeros_like(fp["g"])] * 6)
        args += [fp["W"], cw]
    ct = jnp.zeros((8, 8), jnp.float32).at[0, 0].set(p["pred_target"]["b"][0])
    args += [p["pred_target"]["W"], ct]
    return pl.pallas_call(
        _head_body,
        out_shape=jax.ShapeDtypeStruct((N_GRAPHS, 10), jnp.float32),
    )(*args)


def _conv_layer(node, ea8, src_g, dst_g, dst_s, zeros_acc, p, mean_e, cov_e, dp):
    d = p["bn_g"].shape[0]
    we = _pad2(p["edgemlp"]["W"], 8, dp)
    ge = _pad1(p["edgemlp"]["g"], dp)
    bte = _pad1(p["edgemlp"]["bt"], dp)
    mu_z = mean_e @ we
    var_z = jnp.sum(we * (cov_e @ we), axis=0)
    scale = ge * lax.rsqrt(var_z + EPS)
    shift = bte - mu_z * scale
    c1 = jnp.zeros((8, dp), jnp.float32).at[0].set(scale).at[1].set(shift)

    wm = p["mlp"]["W"]
    wsc = p["screen"]["W"]
    ws = [_pad2(wm[0:d], dp, dp), _pad2(wm[d:2 * d], dp, dp), _pad2(wm[2 * d:], dp, dp),
          _pad2(wsc[0:d], dp, dp), _pad2(wsc[d:2 * d], dp, dp), _pad2(wsc[2 * d:], dp, dp)]

    nsrc = _sc_gather(node, src_g, dp)
    ndst = _sc_gather(node, dst_g, dp)
    ym, ys, mom = _stage1(ea8, nsrc, ndst, ws, we, c1, dp)

    c2 = (jnp.zeros((8, dp), jnp.float32)
          .at[0].set(_pad1(p["mlp"]["g"], dp)).at[1].set(_pad1(p["mlp"]["bt"], dp))
          .at[2].set(_pad1(p["screen"]["g"], dp)).at[3].set(_pad1(p["screen"]["bt"], dp)))
    upd = _stage2(ym, ys, mom, c2, dp)
    acc2 = _sc_scatter_add(upd, dst_s, zeros_acc, dp)
    c3 = jnp.zeros((8, dp), jnp.float32).at[0].set(_pad1(p["bn_g"], dp)).at[1].set(_pad1(p["bn_b"], dp))
    return _node_update(acc2, node, c3, dp)


def _branch(v_in, e_in, edge_index, node_graph, emb_p, convs, d, dp):
    w_emb = _pad2(emb_p["W"], v_in.shape[1], dp)
    c_emb = jnp.zeros((8, dp), jnp.float32).at[0].set(_pad1(emb_p["g"], dp)).at[1].set(_pad1(emb_p["bt"], dp))
    node = _embed(v_in, w_emb, c_emb, dp)

    ea8 = _pad2(e_in, E, 8)
    st = _ea_stats(ea8)
    mean_e = st[8, :] / E
    cov_e = st[0:8, :] / E - jnp.outer(mean_e, mean_e)

    src = edge_index[0].astype(jnp.int32)
    dst = edge_index[1].astype(jnp.int32)
    src_g = jnp.pad(src, (0, E_PAD - E))
    dst_g = jnp.pad(dst, (0, E_PAD - E))
    dst_s = jnp.pad(dst, (0, E_PAD - E), constant_values=N_NODES)
    zeros_acc = jnp.zeros((NROWS, dp), jnp.float32)

    for p in convs:
        node = _conv_layer(node, ea8, src_g, dst_g, dst_s, zeros_acc, p,
                           mean_e, cov_e, dp)

    gid2d = node_graph.astype(jnp.int32).reshape(N_NODES, 1)
    return _pool(node, gid2d, d, dp)


def kernel(va, ea, edge_index_a, node_graph_a, vs, es, edge_index_s,
           node_graph_s, params):
    pa = _branch(va, ea, edge_index_a, node_graph_a, params["emb_a"],
                 params["convs_a"], 110, 112)
    ps = _branch(vs, es, edge_index_s, node_graph_s, params["emb_s"],
                 params["convs_s"], 150, 160)
    vt = jnp.concatenate([pa, ps], axis=1)
    return _head(vt, params)
